# baseline (speedup 1.0000x reference)
import functools

import jax
import jax.numpy as jnp
from jax import lax
from jax.experimental import pallas as pl
from jax.experimental.pallas import tpu as pltpu
from jax.experimental.pallas import tpu_sc as plsc

D_X = 3
N_TOKEN = 48
D_MODEL = 256
L = 16
NG = D_MODEL // L
U_PER_W = 64
U_CHUNK = 2
G_HALF = NG // 2


def _sc_body(x0_hbm, x1_hbm, x2_hbm, wt_hbm, time_hbm, space_hbm, delta_hbm,
             out_hbm, x0_v, x1_v, x2_v, w_v, time_v, space_v, delta_v,
             out_a, out_b, sem_a, sem_b, timesteps):
    nc = 2
    wid = lax.axis_index("s") * nc + lax.axis_index("c")
    u_base = wid * U_PER_W
    t0 = lax.rem(u_base, timesteps)

    pltpu.sync_copy(x0_hbm.at[pl.ds(u_base, U_PER_W)], x0_v)
    pltpu.sync_copy(x1_hbm.at[pl.ds(u_base, U_PER_W)], x1_v)
    pltpu.sync_copy(x2_hbm.at[pl.ds(u_base, U_PER_W)], x2_v)
    pltpu.sync_copy(wt_hbm, w_v)
    pltpu.sync_copy(time_hbm.at[pl.ds(t0, U_PER_W)], time_v)
    pltpu.sync_copy(space_hbm, space_v)
    pltpu.sync_copy(delta_hbm, delta_v)

    bufs = (out_a, out_b)
    sems = (sem_a, sem_b)

    def chunk_pair(it, carry):
        for slot in range(2):
            buf = bufs[slot]
            sem = sems[slot]
            ch = it * 2 + slot

            @pl.when(it > 0)
            def _drain():
                pltpu.make_async_copy(
                    buf, out_hbm.at[pl.ds(0, U_CHUNK)], sem).wait()

            for g_half in range(2):
                sls = [pl.ds((g_half * G_HALF + g) * L, L)
                       for g in range(G_HALF)]
                w0g = [w_v[0, s] for s in sls]
                w1g = [w_v[1, s] for s in sls]
                w2g = [w_v[2, s] for s in sls]
                dg = [delta_v[s] for s in sls]

                def unit(t2, c2, _buf=buf, _ch=ch, _sls=sls,
                         _w0=w0g, _w1=w1g, _w2=w2g, _dg=dg):
                    ul = _ch * U_CHUNK + t2
                    tg = [time_v[ul, s] for s in _sls]

                    def grp(j3, c3):
                        x0g = x0_v[ul, pl.ds(j3 * L, L)]
                        x1g = x1_v[ul, pl.ds(j3 * L, L)]
                        x2g = x2_v[ul, pl.ds(j3 * L, L)]
                        for i in range(L):
                            x0s = x0g[i]
                            x1s = x1g[i]
                            x2s = x2g[i]
                            n0 = x0s != x0s
                            n1 = x1s != x1s
                            n2 = x2s != x2s
                            fl = jnp.where(n0 | n1 | n2, 1.0, 0.0)
                            x0c = jnp.where(n0, 0.0, x0s)
                            x1c = jnp.where(n1, 0.0, x1s)
                            x2c = jnp.where(n2, 0.0, x2s)
                            n = j3 * L + i
                            for g in range(G_HALF):
                                sp = space_v[n, _sls[g]]
                                acc = (x0c * _w0[g] + x1c * _w1[g]
                                       + x2c * _w2[g] + fl * _dg[g]
                                       + tg[g] + sp)
                                _buf[t2, n, _sls[g]] = acc
                        return c3

                    lax.fori_loop(0, N_TOKEN // L, grp, 0)
                    return c2

                lax.fori_loop(0, U_CHUNK, unit, 0)

            pltpu.make_async_copy(
                buf, out_hbm.at[pl.ds(u_base + ch * U_CHUNK, U_CHUNK)],
                sem).start()
        return carry

    n_pairs = U_PER_W // (2 * U_CHUNK)
    lax.fori_loop(0, n_pairs, chunk_pair, 0)

    for slot in range(2):
        pltpu.make_async_copy(
            bufs[slot], out_hbm.at[pl.ds(0, U_CHUNK)], sems[slot]).wait()


def kernel(x, W_lin, b_lin, time_table, space_table, nan_table):
    bsize, timesteps, n_joint, d_joint = x.shape
    n_token = n_joint * d_joint // D_X
    bt = bsize * timesteps
    xr = x.reshape(bt, n_token, D_X)
    x0 = xr[..., 0]
    x1 = xr[..., 1]
    x2 = xr[..., 2]
    wt = W_lin.T
    space2 = space_table + b_lin[None, :] + nan_table[0][None, :]
    delta = nan_table[1] - nan_table[0]

    mesh = plsc.VectorSubcoreMesh(core_axis_name="c", subcore_axis_name="s")
    sc_fn = functools.partial(
        pl.kernel,
        mesh=mesh,
        out_type=jax.ShapeDtypeStruct((bt, n_token, D_MODEL), jnp.float32),
        scratch_types=[
            pltpu.VMEM((U_PER_W, n_token), jnp.float32),
            pltpu.VMEM((U_PER_W, n_token), jnp.float32),
            pltpu.VMEM((U_PER_W, n_token), jnp.float32),
            pltpu.VMEM((D_X, D_MODEL), jnp.float32),
            pltpu.VMEM((U_PER_W, D_MODEL), jnp.float32),
            pltpu.VMEM((n_token, D_MODEL), jnp.float32),
            pltpu.VMEM((D_MODEL,), jnp.float32),
            pltpu.VMEM((U_CHUNK, n_token, D_MODEL), jnp.float32),
            pltpu.VMEM((U_CHUNK, n_token, D_MODEL), jnp.float32),
            pltpu.SemaphoreType.DMA,
            pltpu.SemaphoreType.DMA,
        ],
    )(functools.partial(_sc_body, timesteps=timesteps))
    out = sc_fn(x0, x1, x2, wt, time_table, space2, delta)
    return out.reshape(bsize, timesteps * n_token, D_MODEL)


# speedup vs baseline: 4.8804x; 4.8804x over previous
import jax
import jax.numpy as jnp
from jax.experimental import pallas as pl

D_X = 3
N_TOKEN = 48
T_BLOCK = 256
T_INNER = 1


def _emb_kernel(x0_ref, x1_ref, x2_ref, wt_ref, time_ref,
                space_ref, delta_ref, out_ref):
    w0 = wt_ref[0][None, None, :]
    w1 = wt_ref[1][None, None, :]
    w2 = wt_ref[2][None, None, :]
    delta = delta_ref[0][None, None, :]
    space = space_ref[...][None, :, :]

    nan_any = jnp.any(jnp.isnan(x0_ref[0]) | jnp.isnan(x1_ref[0])
                      | jnp.isnan(x2_ref[0]))

    @pl.when(jnp.logical_not(nan_any))
    def _fast():
        for i in range(T_BLOCK // T_INNER):
            sl = pl.ds(i * T_INNER, T_INNER)
            x0 = x0_ref[0, sl, :]
            x1 = x1_ref[0, sl, :]
            x2 = x2_ref[0, sl, :]
            acc = (x0[:, :, None] * w0 + x1[:, :, None] * w1
                   + x2[:, :, None] * w2)
            out_ref[0, sl, :, :] = acc + time_ref[sl, :][:, None, :] + space

    @pl.when(nan_any)
    def _slow():
        for i in range(T_BLOCK // T_INNER):
            sl = pl.ds(i * T_INNER, T_INNER)
            x0 = x0_ref[0, sl, :]
            x1 = x1_ref[0, sl, :]
            x2 = x2_ref[0, sl, :]
            n0 = jnp.isnan(x0)
            n1 = jnp.isnan(x1)
            n2 = jnp.isnan(x2)
            flag = (n0 | n1 | n2).astype(jnp.float32)
            x0 = jnp.where(n0, 0.0, x0)
            x1 = jnp.where(n1, 0.0, x1)
            x2 = jnp.where(n2, 0.0, x2)
            acc = (x0[:, :, None] * w0 + x1[:, :, None] * w1
                   + x2[:, :, None] * w2 + flag[:, :, None] * delta)
            out_ref[0, sl, :, :] = acc + time_ref[sl, :][:, None, :] + space


def kernel(x, W_lin, b_lin, time_table, space_table, nan_table):
    bsize, timesteps, n_joint, d_joint = x.shape
    n_token = n_joint * d_joint // D_X
    xr = x.reshape(bsize, timesteps, n_token, D_X)
    x0 = xr[..., 0]
    x1 = xr[..., 1]
    x2 = xr[..., 2]
    wt = W_lin.T
    space2 = space_table + b_lin[None, :] + nan_table[0][None, :]
    delta = (nan_table[1] - nan_table[0]).reshape(1, -1)

    d_model = time_table.shape[1]
    grid = (bsize, timesteps // T_BLOCK)
    x_spec = pl.BlockSpec((1, T_BLOCK, n_token), lambda b, j: (b, j, 0))
    out = pl.pallas_call(
        _emb_kernel,
        grid=grid,
        in_specs=[
            x_spec, x_spec, x_spec,
            pl.BlockSpec((D_X, d_model), lambda b, j: (0, 0)),
            pl.BlockSpec((T_BLOCK, d_model), lambda b, j: (j, 0)),
            pl.BlockSpec((n_token, d_model), lambda b, j: (0, 0)),
            pl.BlockSpec((1, d_model), lambda b, j: (0, 0)),
        ],
        out_specs=pl.BlockSpec((1, T_BLOCK, n_token, d_model),
                               lambda b, j: (b, j, 0, 0)),
        out_shape=jax.ShapeDtypeStruct(
            (bsize, timesteps, n_token, d_model), jnp.float32),
    )(x0, x1, x2, wt, time_table, space2, delta)
    return out.reshape(bsize, timesteps * n_token, d_model)


# speedup vs baseline: 4.8868x; 1.0013x over previous
import jax
import jax.numpy as jnp
from jax.experimental import pallas as pl

D_X = 3
N_TOKEN = 48
T_BLOCK = 256
T_INNER = 2


def _emb_kernel(x0_ref, x1_ref, x2_ref, wt_ref, time_ref,
                space_ref, delta_ref, out_ref):
    w0 = wt_ref[0][None, None, :]
    w1 = wt_ref[1][None, None, :]
    w2 = wt_ref[2][None, None, :]
    delta = delta_ref[0][None, None, :]
    space = space_ref[...][None, :, :]

    nan_any = jnp.any(jnp.isnan(x0_ref[0]) | jnp.isnan(x1_ref[0])
                      | jnp.isnan(x2_ref[0]))

    @pl.when(jnp.logical_not(nan_any))
    def _fast():
        for i in range(T_BLOCK // T_INNER):
            sl = pl.ds(i * T_INNER, T_INNER)
            x0 = x0_ref[0, sl, :]
            x1 = x1_ref[0, sl, :]
            x2 = x2_ref[0, sl, :]
            acc = (x0[:, :, None] * w0 + x1[:, :, None] * w1
                   + x2[:, :, None] * w2)
            out_ref[0, sl, :, :] = acc + time_ref[sl, :][:, None, :] + space

    @pl.when(nan_any)
    def _slow():
        for i in range(T_BLOCK // T_INNER):
            sl = pl.ds(i * T_INNER, T_INNER)
            x0 = x0_ref[0, sl, :]
            x1 = x1_ref[0, sl, :]
            x2 = x2_ref[0, sl, :]
            n0 = jnp.isnan(x0)
            n1 = jnp.isnan(x1)
            n2 = jnp.isnan(x2)
            flag = (n0 | n1 | n2).astype(jnp.float32)
            x0 = jnp.where(n0, 0.0, x0)
            x1 = jnp.where(n1, 0.0, x1)
            x2 = jnp.where(n2, 0.0, x2)
            acc = (x0[:, :, None] * w0 + x1[:, :, None] * w1
                   + x2[:, :, None] * w2 + flag[:, :, None] * delta)
            out_ref[0, sl, :, :] = acc + time_ref[sl, :][:, None, :] + space


def kernel(x, W_lin, b_lin, time_table, space_table, nan_table):
    bsize, timesteps, n_joint, d_joint = x.shape
    n_token = n_joint * d_joint // D_X
    xr = x.reshape(bsize, timesteps, n_token, D_X)
    x0 = xr[..., 0]
    x1 = xr[..., 1]
    x2 = xr[..., 2]
    wt = W_lin.T
    space2 = space_table + b_lin[None, :] + nan_table[0][None, :]
    delta = (nan_table[1] - nan_table[0]).reshape(1, -1)

    d_model = time_table.shape[1]
    grid = (bsize, timesteps // T_BLOCK)
    x_spec = pl.BlockSpec((1, T_BLOCK, n_token), lambda b, j: (b, j, 0))
    out = pl.pallas_call(
        _emb_kernel,
        grid=grid,
        in_specs=[
            x_spec, x_spec, x_spec,
            pl.BlockSpec((D_X, d_model), lambda b, j: (0, 0)),
            pl.BlockSpec((T_BLOCK, d_model), lambda b, j: (j, 0)),
            pl.BlockSpec((n_token, d_model), lambda b, j: (0, 0)),
            pl.BlockSpec((1, d_model), lambda b, j: (0, 0)),
        ],
        out_specs=pl.BlockSpec((1, T_BLOCK, n_token, d_model),
                               lambda b, j: (b, j, 0, 0)),
        out_shape=jax.ShapeDtypeStruct(
            (bsize, timesteps, n_token, d_model), jnp.float32),
    )(x0, x1, x2, wt, time_table, space2, delta)
    return out.reshape(bsize, timesteps * n_token, d_model)
